# baseline (speedup 1.0000x reference)
import functools

import jax
import jax.numpy as jnp
from jax import lax
from jax.experimental import pallas as pl
from jax.experimental.pallas import tpu as pltpu
from jax.experimental.pallas import tpu_sc as plsc

_E = 4
_TOPK = 2
_C = 96
_H = 64
_W = 64
_HW = _H * _W
_EPS = 1e-5
_D = 9 * _C
_GROWS = 2 * 8 * _C
_DP = 896


def _sc_gather_kernel(rows_per_w, num_cores, table_hbm, idx_hbm, out_hbm,
                      idx_v, rows_v, sem):
    wid = lax.axis_index("s") * num_cores + lax.axis_index("c")
    base = wid * rows_per_w
    pltpu.sync_copy(idx_hbm.at[pl.ds(base, rows_per_w)], idx_v)
    pltpu.async_copy(table_hbm.at[idx_v], rows_v, sem).wait()
    pltpu.sync_copy(rows_v, out_hbm.at[pl.ds(base, rows_per_w)])


def _moe_conv_kernel(idx_ref, wts_ref,
                     x_ref, w0_ref, w1_ref,
                     g_ref, be_ref, mu_ref, va_ref,
                     out_ref, xcol_ref):
    b = pl.program_id(0)
    xb = x_ref[0]

    n = lax.broadcasted_iota(jnp.int32, (1, _HW), 1)
    hpos = n >> 6
    wpos = n & (_W - 1)
    for t in range(9):
        oy = t // 3 - 1
        ox = t % 3 - 1
        off = oy * _W + ox
        xs = jnp.roll(xb, -off, axis=1) if off != 0 else xb
        mh = (hpos + oy >= 0) & (hpos + oy < _H)
        mw = (wpos + ox >= 0) & (wpos + ox < _W)
        mask = (mh & mw).astype(jnp.bfloat16)
        xcol_ref[t * _C:(t + 1) * _C, :] = xs * mask

    scale_all = g_ref[...] * lax.rsqrt(va_ref[...] + _EPS)
    bias_all = be_ref[...] - mu_ref[...] * scale_all
    erow = lax.broadcasted_iota(jnp.int32, (_E, 1), 0)

    xcol = xcol_ref[...]
    acc = jnp.zeros((_C, _HW), dtype=jnp.float32)
    for slot, w_ref in enumerate((w0_ref, w1_ref)):
        wsel = w_ref[0, 0, :, :_D].astype(jnp.bfloat16)
        y = lax.dot_general(wsel, xcol, (((1,), (0,)), ((), ())),
                            preferred_element_type=jnp.float32)
        onehot = (erow == idx_ref[b, slot]).astype(jnp.float32)
        dims = (((0,), (0,)), ((), ()))
        scale = lax.dot_general(scale_all, onehot, dims,
                                preferred_element_type=jnp.float32)
        bias = lax.dot_general(bias_all, onehot, dims,
                               preferred_element_type=jnp.float32)
        y = y * scale + bias
        y = y * jax.nn.sigmoid(y)
        acc = acc + wts_ref[b, slot] * y
    out_ref[0] = acc.astype(jnp.bfloat16)


def kernel(x, weights, indices, conv_w, bn_gamma, bn_beta, bn_mean, bn_var):
    B = x.shape[0]
    xf = x.reshape(B, _C, _HW).astype(jnp.bfloat16)
    wt = conv_w.transpose(0, 1, 3, 4, 2).reshape(_E * _C, _D)
    wt = jnp.pad(wt, ((0, 0), (0, _DP - _D)))
    idx = indices.astype(jnp.int32)
    wts = weights.astype(jnp.float32)

    gidx = (idx.reshape(B * _TOPK, 1) * _C
            + jnp.arange(_C, dtype=jnp.int32)[None, :]).reshape(_GROWS)
    info = plsc.get_sparse_core_info()
    nw = info.num_cores * info.num_subcores
    rows_per_w = _GROWS // nw
    mesh = plsc.VectorSubcoreMesh(core_axis_name="c", subcore_axis_name="s")
    sc_gather = pl.kernel(
        functools.partial(_sc_gather_kernel, rows_per_w, info.num_cores),
        mesh=mesh,
        out_type=jax.ShapeDtypeStruct((_GROWS, _DP), jnp.float32),
        scratch_types=[
            pltpu.VMEM((rows_per_w,), jnp.int32),
            pltpu.VMEM((rows_per_w, _DP), jnp.float32),
            pltpu.SemaphoreType.DMA,
        ],
    )
    wsel = sc_gather(wt, gidx).reshape(B, _TOPK, _C, _DP)

    grid_spec = pltpu.PrefetchScalarGridSpec(
        num_scalar_prefetch=2,
        grid=(B,),
        in_specs=[
            pl.BlockSpec((1, _C, _HW), lambda b, *_: (b, 0, 0)),
            pl.BlockSpec((1, 1, _C, _DP), lambda b, *_: (b, 0, 0, 0)),
            pl.BlockSpec((1, 1, _C, _DP), lambda b, *_: (b, 1, 0, 0)),
            pl.BlockSpec((_E, _C), lambda b, *_: (0, 0)),
            pl.BlockSpec((_E, _C), lambda b, *_: (0, 0)),
            pl.BlockSpec((_E, _C), lambda b, *_: (0, 0)),
            pl.BlockSpec((_E, _C), lambda b, *_: (0, 0)),
        ],
        out_specs=pl.BlockSpec((1, _C, _HW), lambda b, *_: (b, 0, 0)),
        scratch_shapes=[pltpu.VMEM((9 * _C, _HW), jnp.bfloat16)],
    )

    out = pl.pallas_call(
        _moe_conv_kernel,
        grid_spec=grid_spec,
        out_shape=jax.ShapeDtypeStruct((B, _C, _HW), jnp.bfloat16),
    )(idx, wts, xf, wsel, wsel, bn_gamma, bn_beta, bn_mean, bn_var)
    return out.reshape(B, _C, _H, _W).astype(jnp.float32)


# speedup vs baseline: 1.3006x; 1.3006x over previous
import jax
import jax.numpy as jnp
from jax import lax
from jax.experimental import pallas as pl
from jax.experimental.pallas import tpu as pltpu

_E = 4
_TOPK = 2
_C = 96
_H = 64
_W = 64
_HW = _H * _W
_EPS = 1e-5


def _moe_conv_kernel(idx_ref, wts_ref,
                     x_ref, w0_ref, w1_ref,
                     g_ref, be_ref, mu_ref, va_ref,
                     out_ref, xcol_ref):
    b = pl.program_id(0)
    xb = x_ref[0]

    n = lax.broadcasted_iota(jnp.int32, (1, _HW), 1)
    hpos = n >> 6
    wpos = n & (_W - 1)
    for t in range(9):
        oy = t // 3 - 1
        ox = t % 3 - 1
        off = oy * _W + ox
        xs = jnp.roll(xb, -off, axis=1) if off != 0 else xb
        conds = []
        if oy != 0:
            conds.append((hpos + oy >= 0) & (hpos + oy < _H))
        if ox != 0:
            conds.append((wpos + ox >= 0) & (wpos + ox < _W))
        if conds:
            m = conds[0] if len(conds) == 1 else conds[0] & conds[1]
            xs = xs * m.astype(jnp.bfloat16)
        xcol_ref[t * _C:(t + 1) * _C, :] = xs

    scale_all = g_ref[...] * lax.rsqrt(va_ref[...] + _EPS)
    bias_all = be_ref[...] - mu_ref[...] * scale_all
    erow = lax.broadcasted_iota(jnp.int32, (_E, 1), 0)

    xcol = xcol_ref[...]
    acc = jnp.zeros((_C, _HW), dtype=jnp.float32)
    for slot, w_ref in enumerate((w0_ref, w1_ref)):
        y = lax.dot_general(w_ref[0], xcol, (((1,), (0,)), ((), ())),
                            preferred_element_type=jnp.float32)
        onehot = (erow == idx_ref[b, slot]).astype(jnp.float32)
        dims = (((0,), (0,)), ((), ()))
        scale = lax.dot_general(scale_all, onehot, dims,
                                preferred_element_type=jnp.float32)
        bias = lax.dot_general(bias_all, onehot, dims,
                               preferred_element_type=jnp.float32)
        y = y * scale + bias
        y = y * jax.nn.sigmoid(y)
        acc = acc + wts_ref[b, slot] * y
    out_ref[0] = acc.astype(jnp.bfloat16)


def kernel(x, weights, indices, conv_w, bn_gamma, bn_beta, bn_mean, bn_var):
    B = x.shape[0]
    xf = x.reshape(B, _C, _HW).astype(jnp.bfloat16)
    wf = conv_w.transpose(0, 1, 3, 4, 2).reshape(_E, _C, 9 * _C).astype(jnp.bfloat16)
    idx = indices.astype(jnp.int32)
    wts = weights.astype(jnp.float32)

    def e_map(slot):
        return lambda b, idx_ref, wts_ref: (idx_ref[b, slot], 0, 0)

    grid_spec = pltpu.PrefetchScalarGridSpec(
        num_scalar_prefetch=2,
        grid=(B,),
        in_specs=[
            pl.BlockSpec((1, _C, _HW), lambda b, *_: (b, 0, 0)),
            pl.BlockSpec((1, _C, 9 * _C), e_map(0)),
            pl.BlockSpec((1, _C, 9 * _C), e_map(1)),
            pl.BlockSpec((_E, _C), lambda b, *_: (0, 0)),
            pl.BlockSpec((_E, _C), lambda b, *_: (0, 0)),
            pl.BlockSpec((_E, _C), lambda b, *_: (0, 0)),
            pl.BlockSpec((_E, _C), lambda b, *_: (0, 0)),
        ],
        out_specs=pl.BlockSpec((1, _C, _HW), lambda b, *_: (b, 0, 0)),
        scratch_shapes=[pltpu.VMEM((9 * _C, _HW), jnp.bfloat16)],
    )

    out = pl.pallas_call(
        _moe_conv_kernel,
        grid_spec=grid_spec,
        out_shape=jax.ShapeDtypeStruct((B, _C, _HW), jnp.bfloat16),
    )(idx, wts, xf, wf, wf, bn_gamma, bn_beta, bn_mean, bn_var)
    return out.reshape(B, _C, _H, _W).astype(jnp.float32)


# speedup vs baseline: 1.3533x; 1.0405x over previous
import jax
import jax.numpy as jnp
from jax import lax
from jax.experimental import pallas as pl
from jax.experimental.pallas import tpu as pltpu

_E = 4
_TOPK = 2
_C = 96
_H = 64
_W = 64
_HW = _H * _W
_EPS = 1e-5


def _moe_conv_kernel(idx_ref, wts_ref,
                     x_ref, w0_ref, w1_ref,
                     g_ref, be_ref, mu_ref, va_ref,
                     out_ref, xcol_ref):
    b = pl.program_id(0)
    xb = x_ref[0]

    n = lax.broadcasted_iota(jnp.int32, (1, _HW), 1)
    hpos = n >> 6
    wpos = n & (_W - 1)
    for t in range(9):
        oy = t // 3 - 1
        ox = t % 3 - 1
        off = oy * _W + ox
        xs = jnp.roll(xb, -off, axis=1) if off != 0 else xb
        conds = []
        if oy != 0:
            conds.append((hpos + oy >= 0) & (hpos + oy < _H))
        if ox != 0:
            conds.append((wpos + ox >= 0) & (wpos + ox < _W))
        if conds:
            m = conds[0] if len(conds) == 1 else conds[0] & conds[1]
            xs = xs * m.astype(jnp.bfloat16)
        xcol_ref[t * _C:(t + 1) * _C, :] = xs

    scale_all = g_ref[...] * lax.rsqrt(va_ref[...] + _EPS)
    bias_all = be_ref[...] - mu_ref[...] * scale_all
    erow = lax.broadcasted_iota(jnp.int32, (_E, 1), 0)

    xcol = xcol_ref[...]
    wcat = jnp.concatenate([w0_ref[0], w1_ref[0]], axis=0)
    z = lax.dot_general(wcat, xcol, (((1,), (0,)), ((), ())),
                        preferred_element_type=jnp.float32)
    acc = jnp.zeros((_C, _HW), dtype=jnp.float32)
    for slot in range(2):
        y = z[slot * _C:(slot + 1) * _C, :]
        onehot = (erow == idx_ref[b, slot]).astype(jnp.float32)
        dims = (((0,), (0,)), ((), ()))
        scale = lax.dot_general(scale_all, onehot, dims,
                                preferred_element_type=jnp.float32)
        bias = lax.dot_general(bias_all, onehot, dims,
                               preferred_element_type=jnp.float32)
        y = y * scale + bias
        y = y * jax.nn.sigmoid(y)
        acc = acc + wts_ref[b, slot] * y
    out_ref[0] = acc.astype(jnp.bfloat16)


def kernel(x, weights, indices, conv_w, bn_gamma, bn_beta, bn_mean, bn_var):
    B = x.shape[0]
    xf = x.reshape(B, _C, _HW).astype(jnp.bfloat16)
    wf = conv_w.transpose(0, 1, 3, 4, 2).reshape(_E, _C, 9 * _C).astype(jnp.bfloat16)
    idx = indices.astype(jnp.int32)
    wts = weights.astype(jnp.float32)

    def e_map(slot):
        return lambda b, idx_ref, wts_ref: (idx_ref[b, slot], 0, 0)

    grid_spec = pltpu.PrefetchScalarGridSpec(
        num_scalar_prefetch=2,
        grid=(B,),
        in_specs=[
            pl.BlockSpec((1, _C, _HW), lambda b, *_: (b, 0, 0)),
            pl.BlockSpec((1, _C, 9 * _C), e_map(0)),
            pl.BlockSpec((1, _C, 9 * _C), e_map(1)),
            pl.BlockSpec((_E, _C), lambda b, *_: (0, 0)),
            pl.BlockSpec((_E, _C), lambda b, *_: (0, 0)),
            pl.BlockSpec((_E, _C), lambda b, *_: (0, 0)),
            pl.BlockSpec((_E, _C), lambda b, *_: (0, 0)),
        ],
        out_specs=pl.BlockSpec((1, _C, _HW), lambda b, *_: (b, 0, 0)),
        scratch_shapes=[pltpu.VMEM((9 * _C, _HW), jnp.bfloat16)],
    )

    out = pl.pallas_call(
        _moe_conv_kernel,
        grid_spec=grid_spec,
        out_shape=jax.ShapeDtypeStruct((B, _C, _HW), jnp.bfloat16),
    )(idx, wts, xf, wf, wf, bn_gamma, bn_beta, bn_mean, bn_var)
    return out.reshape(B, _C, _H, _W).astype(jnp.float32)
